# baseline (speedup 1.0000x reference)
import functools

import jax
import jax.numpy as jnp
from jax import lax
from jax.experimental import pallas as pl
from jax.experimental.pallas import tpu as pltpu
from jax.experimental.pallas import tpu_sc as plsc

_PCALL = pl.pallas_call

N = 10000
D = 128
NH = 4
HC = 32
R = 256
CB = 2048
NPAD = 10240
NCH = NPAD // CB
GRID = (N + R - 1) // R
IMAX = 2**31 - 1

SC_W = 32
SC_CH = 6
SC_G = 4
SC_GROUPS = 14
SC_PER_W = SC_CH * SC_G * SC_GROUPS
NSC = SC_W * SC_PER_W

_HIGH = jax.lax.Precision.HIGHEST


def _dot(a, b):
    return lax.dot_general(a, b, (((1,), (0,)), ((), ())),
                           preferred_element_type=jnp.float32,
                           precision=_HIGH)


def _elu(v):
    return jnp.where(v > 0.0, v, jnp.exp(v) - 1.0)


def _d2_chunk(pb, p2i, post_ref, c):
    pj = post_ref[0:3, c * CB:(c + 1) * CB]
    pjf = pj.astype(jnp.float32)
    p2j = jnp.sum(pjf * pjf, axis=0, keepdims=True)
    dot = lax.dot_general(pb, pj, (((1,), (0,)), ((), ())),
                          preferred_element_type=jnp.float32)
    return p2i + p2j - 2.0 * dot


def _extract_topk(key, k, width):
    buf = jnp.full((key.shape[0], width), IMAX, jnp.int32)
    lane = lax.broadcasted_iota(jnp.int32, (key.shape[0], width), 1)
    for t in range(k):
        m = jnp.min(key, axis=1, keepdims=True)
        buf = jnp.where(lane == t, m, buf)
        key = jnp.where(key == m, IMAX, key)
    return buf


SUBW = 256
NSUB = NPAD // SUBW
SUBK = 8


def _kernelA_body(x_ref, pos_ref, post_ref, wg_ref, wa_ref, asrc_ref,
                  adst_ref, nbr_out, y_out, dinv_out, xwa_out, esrc_out,
                  edst_out, key_ref):
    i = pl.program_id(0)
    x = x_ref[...]
    xwg = _dot(x, wg_ref[...])
    xwa = _dot(x, wa_ref[...])
    xwa_out[...] = xwa
    esrc_out[...] = jnp.exp(_dot(xwa, asrc_ref[...]))
    edst_out[...] = jnp.exp(_dot(xwa, adst_ref[...]))

    p = pos_ref[...]
    pb = p.astype(jnp.bfloat16)
    p2i = jnp.sum(p * p, axis=1, keepdims=True)
    rows = i * R + lax.broadcasted_iota(jnp.int32, (R, 1), 0)

    deg = jnp.zeros((R, 1), jnp.float32)
    for c in range(NCH):
        d2 = _d2_chunk(pb, p2i, post_ref, c)
        deg = deg + jnp.sum(jnp.where(d2 <= 36.0, 1.0, 0.0),
                            axis=1, keepdims=True)
        jj = c * CB + lax.broadcasted_iota(jnp.int32, (R, CB), 1)
        valid = (d2 < 32768.0) & (jj != rows)
        key_ref[:, c * CB:(c + 1) * CB] = jnp.where(
            valid, d2.astype(jnp.int32) * 16384 + jj, IMAX)

    bufs = []
    for s in range(NSUB):
        bufs.append(_extract_topk(key_ref[:, s * SUBW:(s + 1) * SUBW],
                                  SUBK, SUBK))
    cands = jnp.concatenate(bufs, axis=1)
    top = _extract_topk(cands, 20, 32)
    t20 = top[:, 19:20]
    cnt = jnp.zeros((R, 1), jnp.int32)
    for c in range(NCH):
        k = key_ref[:, c * CB:(c + 1) * CB]
        cnt = cnt + jnp.sum(jnp.where(k <= t20, 1, 0), axis=1, keepdims=True)
    bad = jnp.sum(jnp.where(cnt != 20, 1, 0)) > 0

    def _full():
        bs = [_extract_topk(key_ref[:, c * CB:(c + 1) * CB], 20, 32)
              for c in range(NCH)]
        return _extract_topk(jnp.concatenate(bs, axis=1), 20, 32)

    top = lax.cond(bad, _full, lambda: top)
    lane = lax.broadcasted_iota(jnp.int32, (R, 32), 1)
    nbr_out[...] = jnp.where(lane < 20, jnp.bitwise_and(top, 16383), 0)

    dinv = lax.rsqrt(deg)
    dinvb = jnp.broadcast_to(dinv, (R, D))
    dinv_out[...] = dinvb
    y_out[...] = dinvb * xwg


def _kernelB_body(pos_ref, edst_ref, dinv_ref, bgcn_ref, bgat_ref,
                  post_ref, esrct_ref, y_ref, xwa_ref, z1_out, z2_out):
    p = pos_ref[...]
    pb = p.astype(jnp.bfloat16)
    p2i = jnp.sum(p * p, axis=1, keepdims=True)

    acc1 = jnp.zeros((R, D), jnp.float32)
    acc2 = [jnp.zeros((R, HC), jnp.float32) for _ in range(NH)]
    ssum = [jnp.zeros((R, 1), jnp.float32) for _ in range(NH)]
    for c in range(NCH):
        d2 = _d2_chunk(pb, p2i, post_ref, c)
        a = jnp.where(d2 <= 36.0, 1.0, 0.0).astype(jnp.bfloat16)
        acc1 = acc1 + lax.dot_general(
            a, y_ref[c * CB:(c + 1) * CB, :], (((1,), (0,)), ((), ())),
            preferred_element_type=jnp.float32)
        mm = d2 <= 100.0
        for h in range(NH):
            eai = edst_ref[:, h:h + 1]
            ea2i = edst_ref[:, 4 + h:5 + h]
            ebj = esrct_ref[h:h + 1, c * CB:(c + 1) * CB]
            eb2j = esrct_ref[4 + h:5 + h, c * CB:(c + 1) * CB]
            p1 = eai * ebj
            p2 = ea2i * eb2j
            ex = jnp.where(mm, jnp.where(p1 > 1.0, p1, p2), 0.0)
            ssum[h] = ssum[h] + jnp.sum(ex, axis=1, keepdims=True)
            acc2[h] = acc2[h] + lax.dot_general(
                ex.astype(jnp.bfloat16),
                xwa_ref[c * CB:(c + 1) * CB, h * HC:(h + 1) * HC],
                (((1,), (0,)), ((), ())),
                preferred_element_type=jnp.float32)
    z1_out[...] = _elu(dinv_ref[...] * acc1 + bgcn_ref[0:1, :])
    parts = [acc2[h] / (ssum[h] + 1e-16) for h in range(NH)]
    z2_out[...] = _elu(jnp.concatenate(parts, axis=1) + bgat_ref[0:1, :])


def _kernelC_body(z1_ref, z2_ref, xsum_ref, x_ref, wl_ref, wr_ref, bl_ref,
                  wg_ref, bg_ref, gamma_ref, beta_ref, z3_out, zn_out):
    mean = xsum_ref[...] / 20.0
    z3 = _elu(_dot(mean, wl_ref[...]) + bl_ref[0:1, :]
              + _dot(x_ref[...], wr_ref[...]))
    z3_out[...] = z3
    z1 = z1_ref[...]
    z2 = z2_ref[...]
    logits = (_dot(z1, wg_ref[0:D, :]) + _dot(z2, wg_ref[D:2 * D, :])
              + _dot(z3, wg_ref[2 * D:3 * D, :]) + bg_ref[0:1, :])
    lane = lax.broadcasted_iota(jnp.int32, logits.shape, 1)
    lm = jnp.max(jnp.where(lane < 3, logits, -1e30), axis=1, keepdims=True)
    exl = jnp.where(lane < 3, jnp.exp(logits - lm), 0.0)
    g = exl / jnp.sum(exl, axis=1, keepdims=True)
    zf = g[:, 0:1] * z1 + g[:, 1:2] * z2 + g[:, 2:3] * z3
    mu = jnp.mean(zf, axis=1, keepdims=True)
    var = jnp.mean((zf - mu) ** 2, axis=1, keepdims=True)
    zn_out[...] = ((zf - mu) / jnp.sqrt(var + 1e-5) * gamma_ref[0:1, :]
                   + beta_ref[0:1, :])


def _build_sc_gather():
    mesh = plsc.VectorSubcoreMesh(core_axis_name="c", subcore_axis_name="s")

    @functools.partial(
        pl.kernel, mesh=mesh,
        out_type=jax.ShapeDtypeStruct((NSC, D), jnp.float32),
        scratch_types=[
            pltpu.VMEM((SC_CH * 20,), jnp.int32),
            pltpu.VMEM((SC_CH * 20, D), jnp.float32),
            pltpu.VMEM((SC_CH * SC_G, D), jnp.float32),
            pltpu.SemaphoreType.DMA,
        ],
    )
    def sc_gather(nbr_hbm, x_hbm, out_hbm, idx_v, rows_v, acc_v, sem):
        wid = lax.axis_index("s") * 2 + lax.axis_index("c")

        def group_body(grp, _):
            gbase = wid * SC_PER_W + grp * (SC_CH * SC_G)
            for g in range(SC_G):
                base = gbase + g * SC_CH
                pltpu.sync_copy(nbr_hbm.at[pl.ds(base * 20, SC_CH * 20)],
                                idx_v)
                pltpu.async_copy(x_hbm.at[idx_v], rows_v, sem).wait()

                def node_body(n, __):
                    for cc in range(D // 16):
                        sl = pl.ds(cc * 16, 16)
                        acc = rows_v[n * 20, sl]
                        for k in range(1, 20):
                            acc = acc + rows_v[n * 20 + k, sl]
                        acc_v[g * SC_CH + n, sl] = acc
                    return __

                lax.fori_loop(0, SC_CH, node_body, 0)
            pltpu.sync_copy(acc_v, out_hbm.at[pl.ds(gbase, SC_CH * SC_G)])
            return _

        lax.fori_loop(0, SC_GROUPS, group_body, 0)

    return sc_gather


_SC_GATHER = _build_sc_gather()


def _att_matrix(att, negslope=0.2):
    eye = jnp.eye(NH, dtype=jnp.float32)
    blk = (att[:, :, None] * eye[:, None, :]).reshape(D, NH)
    return jnp.concatenate([blk, negslope * blk], axis=1)


def kernel(x, pos, W_gcn, b_gcn, W_gat, att_src, att_dst, b_gat, W_sage_l,
           b_sage_l, W_sage_r, W_gate, b_gate, gamma, beta):
    f32 = jnp.float32
    posT = jnp.pad(pos.T, ((0, 5), (0, NPAD - N)),
                   constant_values=1024.0).astype(jnp.bfloat16)
    asrc_m = _att_matrix(att_src)
    adst_m = _att_matrix(att_dst)

    full = lambda shape: pl.BlockSpec(shape, lambda i: tuple(0 for _ in shape))
    rblk = lambda d: pl.BlockSpec((R, d), lambda i: (i, 0))

    nbr, y, dinv, xwa, esrc, edst = _PCALL(
        _kernelA_body,
        grid=GRID,
        in_specs=[rblk(D), rblk(3), full((8, NPAD)), full((D, D)),
                  full((D, D)), full((D, 8)), full((D, 8))],
        out_specs=[rblk(32), rblk(D), rblk(D), rblk(D), rblk(8), rblk(8)],
        out_shape=[
            jax.ShapeDtypeStruct((N, 32), jnp.int32),
            jax.ShapeDtypeStruct((N, D), f32),
            jax.ShapeDtypeStruct((N, D), f32),
            jax.ShapeDtypeStruct((N, D), f32),
            jax.ShapeDtypeStruct((N, 8), f32),
            jax.ShapeDtypeStruct((N, 8), f32),
        ],
        scratch_shapes=[pltpu.VMEM((R, NPAD), jnp.int32)],
    )(x, pos, posT, W_gcn, W_gat, asrc_m, adst_m)

    nbr_flat = jnp.pad(nbr[:, :20], ((0, NSC - N), (0, 0))).reshape(-1)
    xsum = _SC_GATHER(nbr_flat, x)[:N]

    ypad = jnp.pad(y, ((0, NPAD - N), (0, 0))).astype(jnp.bfloat16)
    xwapad = jnp.pad(xwa, ((0, NPAD - N), (0, 0))).astype(jnp.bfloat16)
    esrcT = jnp.pad(esrc.T, ((0, 0), (0, NPAD - N)))

    bgcn8 = jnp.broadcast_to(b_gcn[None, :], (8, D))
    bgat8 = jnp.broadcast_to(b_gat[None, :], (8, D))

    z1, z2 = _PCALL(
        _kernelB_body,
        grid=GRID,
        in_specs=[rblk(3), rblk(8), rblk(D), full((8, D)), full((8, D)),
                  full((8, NPAD)), full((8, NPAD)), full((NPAD, D)),
                  full((NPAD, D))],
        out_specs=[rblk(D), rblk(D)],
        out_shape=[jax.ShapeDtypeStruct((N, D), f32),
                   jax.ShapeDtypeStruct((N, D), f32)],
    )(pos, edst, dinv, bgcn8, bgat8, posT, esrcT, ypad, xwapad)

    bl8 = jnp.broadcast_to(b_sage_l[None, :], (8, D))
    wg_pad = jnp.pad(W_gate, ((0, 0), (0, D - 3)))
    bg8 = jnp.broadcast_to(jnp.pad(b_gate, (0, D - 3))[None, :], (8, D))
    gamma8 = jnp.broadcast_to(gamma[None, :], (8, D))
    beta8 = jnp.broadcast_to(beta[None, :], (8, D))

    z3, zn = _PCALL(
        _kernelC_body,
        grid=GRID,
        in_specs=[rblk(D), rblk(D), rblk(D), rblk(D), full((D, D)),
                  full((D, D)), full((8, D)), full((3 * D, D)),
                  full((8, D)), full((8, D)), full((8, D))],
        out_specs=[rblk(D), rblk(D)],
        out_shape=[jax.ShapeDtypeStruct((N, D), f32),
                   jax.ShapeDtypeStruct((N, D), f32)],
    )(z1, z2, xsum, x, W_sage_l, W_sage_r, bl8, wg_pad, bg8, gamma8, beta8)

    return ((z1, z2, z3), zn)


# speedup vs baseline: 1.3048x; 1.3048x over previous
import functools

import jax
import jax.numpy as jnp
from jax import lax
from jax.experimental import pallas as pl
from jax.experimental.pallas import tpu as pltpu
from jax.experimental.pallas import tpu_sc as plsc

_PCALL = pl.pallas_call

N = 10000
D = 128
NH = 4
HC = 32
R = 256
CB = 2048
NPAD = 10240
NCH = NPAD // CB
GRID = (N + R - 1) // R
IMAX = 2**31 - 1

SC_W = 32
SC_CH = 6
SC_G = 4
SC_GROUPS = 14
SC_PER_W = SC_CH * SC_G * SC_GROUPS
NSC = SC_W * SC_PER_W

_HIGH = jax.lax.Precision.HIGHEST


def _dot(a, b):
    return lax.dot_general(a, b, (((1,), (0,)), ((), ())),
                           preferred_element_type=jnp.float32,
                           precision=_HIGH)


def _elu(v):
    return jnp.where(v > 0.0, v, jnp.exp(v) - 1.0)


def _d2_chunk(pb, p2i, post_ref, c):
    pj = post_ref[0:3, c * CB:(c + 1) * CB]
    pjf = pj.astype(jnp.float32)
    p2j = jnp.sum(pjf * pjf, axis=0, keepdims=True)
    dot = lax.dot_general(pb, pj, (((1,), (0,)), ((), ())),
                          preferred_element_type=jnp.float32)
    return p2i + p2j - 2.0 * dot


def _extract_topk(key, k, width):
    buf = jnp.full((key.shape[0], width), IMAX, jnp.int32)
    lane = lax.broadcasted_iota(jnp.int32, (key.shape[0], width), 1)
    for t in range(k):
        m = jnp.min(key, axis=1, keepdims=True)
        buf = jnp.where(lane == t, m, buf)
        key = jnp.where(key == m, IMAX, key)
    return buf


def _kernelA_body(x_ref, pos_ref, post_ref, wg_ref, wa_ref, asrc_ref,
                  adst_ref, nbr_out, y_out, dinv_out, xwa_out, esrc_out,
                  edst_out):
    i = pl.program_id(0)
    x = x_ref[...]
    xwg = _dot(x, wg_ref[...])
    xwa = _dot(x, wa_ref[...])
    xwa_out[...] = xwa
    esrc_out[...] = jnp.exp(_dot(xwa, asrc_ref[...]))
    edst_out[...] = jnp.exp(_dot(xwa, adst_ref[...]))

    p = pos_ref[...]
    pb = p.astype(jnp.bfloat16)
    p2i = jnp.sum(p * p, axis=1, keepdims=True)
    rows = i * R + lax.broadcasted_iota(jnp.int32, (R, 1), 0)

    deg = jnp.zeros((R, 1), jnp.float32)
    bufs = []
    for c in range(NCH):
        d2 = _d2_chunk(pb, p2i, post_ref, c)
        deg = deg + jnp.sum(jnp.where(d2 <= 36.0, 1.0, 0.0),
                            axis=1, keepdims=True)
        jj = c * CB + lax.broadcasted_iota(jnp.int32, (R, CB), 1)
        valid = (d2 < 32768.0) & (jj != rows)
        key = jnp.where(valid, d2.astype(jnp.int32) * 16384 + jj, IMAX)
        bufs.append(_extract_topk(key, 20, 32))
    allk = jnp.concatenate(bufs, axis=1)
    top = _extract_topk(allk, 20, 32)
    lane = lax.broadcasted_iota(jnp.int32, (R, 32), 1)
    nbr_out[...] = jnp.where(lane < 20, jnp.bitwise_and(top, 16383), 0)

    dinv = lax.rsqrt(deg)
    dinvb = jnp.broadcast_to(dinv, (R, D))
    dinv_out[...] = dinvb
    y_out[...] = dinvb * xwg


def _kernelB_body(pos_ref, edst_ref, dinv_ref, bgcn_ref, bgat_ref,
                  post_ref, esrct_ref, y_ref, xwa_ref, z1_out, z2_out):
    p = pos_ref[...]
    pb = p.astype(jnp.bfloat16)
    p2i = jnp.sum(p * p, axis=1, keepdims=True)

    acc1 = jnp.zeros((R, D), jnp.float32)
    acc2 = [jnp.zeros((R, HC), jnp.float32) for _ in range(NH)]
    ssum = [jnp.zeros((R, 1), jnp.float32) for _ in range(NH)]
    for c in range(NCH):
        d2 = _d2_chunk(pb, p2i, post_ref, c)
        a = jnp.where(d2 <= 36.0, 1.0, 0.0).astype(jnp.bfloat16)
        acc1 = acc1 + lax.dot_general(
            a, y_ref[c * CB:(c + 1) * CB, :], (((1,), (0,)), ((), ())),
            preferred_element_type=jnp.float32)
        mm = d2 <= 100.0
        for h in range(NH):
            eai = edst_ref[:, h:h + 1]
            ea2i = edst_ref[:, 4 + h:5 + h]
            ebj = esrct_ref[h:h + 1, c * CB:(c + 1) * CB]
            eb2j = esrct_ref[4 + h:5 + h, c * CB:(c + 1) * CB]
            ex = jnp.where(mm, jnp.maximum(eai * ebj, ea2i * eb2j), 0.0)
            ssum[h] = ssum[h] + jnp.sum(ex, axis=1, keepdims=True)
            acc2[h] = acc2[h] + lax.dot_general(
                ex.astype(jnp.bfloat16),
                xwa_ref[c * CB:(c + 1) * CB, h * HC:(h + 1) * HC],
                (((1,), (0,)), ((), ())),
                preferred_element_type=jnp.float32)
    z1_out[...] = _elu(dinv_ref[...] * acc1 + bgcn_ref[0:1, :])
    parts = [acc2[h] / (ssum[h] + 1e-16) for h in range(NH)]
    z2_out[...] = _elu(jnp.concatenate(parts, axis=1) + bgat_ref[0:1, :])


def _kernelC_body(z1_ref, z2_ref, xsum_ref, x_ref, wl_ref, wr_ref, bl_ref,
                  wg_ref, bg_ref, gamma_ref, beta_ref, z3_out, zn_out):
    mean = xsum_ref[...] / 20.0
    z3 = _elu(_dot(mean, wl_ref[...]) + bl_ref[0:1, :]
              + _dot(x_ref[...], wr_ref[...]))
    z3_out[...] = z3
    z1 = z1_ref[...]
    z2 = z2_ref[...]
    logits = (_dot(z1, wg_ref[0:D, :]) + _dot(z2, wg_ref[D:2 * D, :])
              + _dot(z3, wg_ref[2 * D:3 * D, :]) + bg_ref[0:1, :])
    lane = lax.broadcasted_iota(jnp.int32, logits.shape, 1)
    lm = jnp.max(jnp.where(lane < 3, logits, -1e30), axis=1, keepdims=True)
    exl = jnp.where(lane < 3, jnp.exp(logits - lm), 0.0)
    g = exl / jnp.sum(exl, axis=1, keepdims=True)
    zf = g[:, 0:1] * z1 + g[:, 1:2] * z2 + g[:, 2:3] * z3
    mu = jnp.mean(zf, axis=1, keepdims=True)
    var = jnp.mean((zf - mu) ** 2, axis=1, keepdims=True)
    zn_out[...] = ((zf - mu) / jnp.sqrt(var + 1e-5) * gamma_ref[0:1, :]
                   + beta_ref[0:1, :])


def _build_sc_gather():
    mesh = plsc.VectorSubcoreMesh(core_axis_name="c", subcore_axis_name="s")

    @functools.partial(
        pl.kernel, mesh=mesh,
        out_type=jax.ShapeDtypeStruct((NSC, D), jnp.float32),
        scratch_types=[
            pltpu.VMEM((SC_CH * 20,), jnp.int32),
            pltpu.VMEM((SC_CH * 20, D), jnp.float32),
            pltpu.VMEM((SC_CH * SC_G, D), jnp.float32),
            pltpu.SemaphoreType.DMA,
        ],
    )
    def sc_gather(nbr_hbm, x_hbm, out_hbm, idx_v, rows_v, acc_v, sem):
        wid = lax.axis_index("s") * 2 + lax.axis_index("c")

        def group_body(grp, _):
            gbase = wid * SC_PER_W + grp * (SC_CH * SC_G)
            for g in range(SC_G):
                base = gbase + g * SC_CH
                pltpu.sync_copy(nbr_hbm.at[pl.ds(base * 20, SC_CH * 20)],
                                idx_v)
                pltpu.async_copy(x_hbm.at[idx_v], rows_v, sem).wait()

                def node_body(n, __):
                    for cc in range(D // 16):
                        sl = pl.ds(cc * 16, 16)
                        acc = rows_v[n * 20, sl]
                        for k in range(1, 20):
                            acc = acc + rows_v[n * 20 + k, sl]
                        acc_v[g * SC_CH + n, sl] = acc
                    return __

                lax.fori_loop(0, SC_CH, node_body, 0)
            pltpu.sync_copy(acc_v, out_hbm.at[pl.ds(gbase, SC_CH * SC_G)])
            return _

        lax.fori_loop(0, SC_GROUPS, group_body, 0)

    return sc_gather


_SC_GATHER = _build_sc_gather()


def _att_matrix(att, negslope=0.2):
    eye = jnp.eye(NH, dtype=jnp.float32)
    blk = (att[:, :, None] * eye[:, None, :]).reshape(D, NH)
    return jnp.concatenate([blk, negslope * blk], axis=1)


def kernel(x, pos, W_gcn, b_gcn, W_gat, att_src, att_dst, b_gat, W_sage_l,
           b_sage_l, W_sage_r, W_gate, b_gate, gamma, beta):
    f32 = jnp.float32
    posT = jnp.pad(pos.T, ((0, 5), (0, NPAD - N)),
                   constant_values=1024.0).astype(jnp.bfloat16)
    asrc_m = _att_matrix(att_src)
    adst_m = _att_matrix(att_dst)

    full = lambda shape: pl.BlockSpec(shape, lambda i: tuple(0 for _ in shape))
    rblk = lambda d: pl.BlockSpec((R, d), lambda i: (i, 0))

    nbr, y, dinv, xwa, esrc, edst = _PCALL(
        _kernelA_body,
        grid=GRID,
        in_specs=[rblk(D), rblk(3), full((8, NPAD)), full((D, D)),
                  full((D, D)), full((D, 8)), full((D, 8))],
        out_specs=[rblk(32), rblk(D), rblk(D), rblk(D), rblk(8), rblk(8)],
        out_shape=[
            jax.ShapeDtypeStruct((N, 32), jnp.int32),
            jax.ShapeDtypeStruct((N, D), f32),
            jax.ShapeDtypeStruct((N, D), f32),
            jax.ShapeDtypeStruct((N, D), f32),
            jax.ShapeDtypeStruct((N, 8), f32),
            jax.ShapeDtypeStruct((N, 8), f32),
        ],
    )(x, pos, posT, W_gcn, W_gat, asrc_m, adst_m)

    nbr_flat = jnp.pad(nbr[:, :20], ((0, NSC - N), (0, 0))).reshape(-1)
    xsum = _SC_GATHER(nbr_flat, x)[:N]

    ypad = jnp.pad(y, ((0, NPAD - N), (0, 0))).astype(jnp.bfloat16)
    xwapad = jnp.pad(xwa, ((0, NPAD - N), (0, 0))).astype(jnp.bfloat16)
    esrcT = jnp.pad(esrc.T, ((0, 0), (0, NPAD - N)))

    bgcn8 = jnp.broadcast_to(b_gcn[None, :], (8, D))
    bgat8 = jnp.broadcast_to(b_gat[None, :], (8, D))

    z1, z2 = _PCALL(
        _kernelB_body,
        grid=GRID,
        in_specs=[rblk(3), rblk(8), rblk(D), full((8, D)), full((8, D)),
                  full((8, NPAD)), full((8, NPAD)), full((NPAD, D)),
                  full((NPAD, D))],
        out_specs=[rblk(D), rblk(D)],
        out_shape=[jax.ShapeDtypeStruct((N, D), f32),
                   jax.ShapeDtypeStruct((N, D), f32)],
    )(pos, edst, dinv, bgcn8, bgat8, posT, esrcT, ypad, xwapad)

    bl8 = jnp.broadcast_to(b_sage_l[None, :], (8, D))
    wg_pad = jnp.pad(W_gate, ((0, 0), (0, D - 3)))
    bg8 = jnp.broadcast_to(jnp.pad(b_gate, (0, D - 3))[None, :], (8, D))
    gamma8 = jnp.broadcast_to(gamma[None, :], (8, D))
    beta8 = jnp.broadcast_to(beta[None, :], (8, D))

    z3, zn = _PCALL(
        _kernelC_body,
        grid=GRID,
        in_specs=[rblk(D), rblk(D), rblk(D), rblk(D), full((D, D)),
                  full((D, D)), full((8, D)), full((3 * D, D)),
                  full((8, D)), full((8, D)), full((8, D))],
        out_specs=[rblk(D), rblk(D)],
        out_shape=[jax.ShapeDtypeStruct((N, D), f32),
                   jax.ShapeDtypeStruct((N, D), f32)],
    )(z1, z2, xsum, x, W_sage_l, W_sage_r, bl8, wg_pad, bg8, gamma8, beta8)

    return ((z1, z2, z3), zn)


# speedup vs baseline: 1.3757x; 1.0543x over previous
import functools

import jax
import jax.numpy as jnp
from jax import lax
from jax.experimental import pallas as pl
from jax.experimental.pallas import tpu as pltpu
from jax.experimental.pallas import tpu_sc as plsc

_PCALL = pl.pallas_call

N = 10000
D = 128
NH = 4
HC = 32
R = 256
CB = 2048
NPAD = 10240
NCH = NPAD // CB
GRID = (N + R - 1) // R
IMAX = 2**31 - 1

SC_W = 32
SC_CH = 6
SC_G = 4
SC_GROUPS = 14
SC_PER_W = SC_CH * SC_G * SC_GROUPS
NSC = SC_W * SC_PER_W

_HIGH = jax.lax.Precision.HIGHEST


def _dot(a, b):
    return lax.dot_general(a, b, (((1,), (0,)), ((), ())),
                           preferred_element_type=jnp.float32,
                           precision=_HIGH)


def _elu(v):
    return jnp.where(v > 0.0, v, jnp.exp(v) - 1.0)


def _d2_chunk(pb, p2i, post_ref, c):
    pj = post_ref[0:3, c * CB:(c + 1) * CB]
    pjf = pj.astype(jnp.float32)
    p2j = jnp.sum(pjf * pjf, axis=0, keepdims=True)
    dot = lax.dot_general(pb, pj, (((1,), (0,)), ((), ())),
                          preferred_element_type=jnp.float32)
    return p2i + p2j - 2.0 * dot


def _extract_topk(key, k, width):
    buf = jnp.full((key.shape[0], width), IMAX, jnp.int32)
    lane = lax.broadcasted_iota(jnp.int32, (key.shape[0], width), 1)
    for t in range(k):
        m = jnp.min(key, axis=1, keepdims=True)
        buf = jnp.where(lane == t, m, buf)
        key = jnp.where(key == m, IMAX, key)
    return buf


def _kernelA_body(x_ref, pos_ref, post_ref, wg_ref, wa_ref, asrc_ref,
                  adst_ref, nbr_out, y_out, dinv_out, xwa_out, esrc_out,
                  edst_out):
    i = pl.program_id(0)
    x = x_ref[...]
    xwg = _dot(x, wg_ref[...])
    xwa = _dot(x, wa_ref[...])
    xwa_out[...] = xwa
    esrc_out[...] = jnp.exp(_dot(xwa, asrc_ref[...]))
    edst_out[...] = jnp.exp(_dot(xwa, adst_ref[...]))

    p = pos_ref[...]
    pb = p.astype(jnp.bfloat16)
    p2i = jnp.sum(p * p, axis=1, keepdims=True)
    rows = i * R + lax.broadcasted_iota(jnp.int32, (R, 1), 0)

    deg = jnp.zeros((R, 1), jnp.float32)
    bufs = []
    for c in range(NCH):
        d2 = _d2_chunk(pb, p2i, post_ref, c)
        deg = deg + jnp.sum(jnp.where(d2 <= 36.0, 1.0, 0.0),
                            axis=1, keepdims=True)
        jj = c * CB + lax.broadcasted_iota(jnp.int32, (R, CB), 1)
        valid = (d2 < 32768.0) & (jj != rows)
        bufs.append(jnp.where(valid, d2.astype(jnp.int32) * 16384 + jj,
                              IMAX))
    allk = jnp.concatenate(bufs, axis=1)
    top = _extract_topk(allk, 20, 32)
    lane = lax.broadcasted_iota(jnp.int32, (R, 32), 1)
    nbr_out[...] = jnp.where(lane < 20, jnp.bitwise_and(top, 16383), 0)

    dinv = lax.rsqrt(deg)
    dinvb = jnp.broadcast_to(dinv, (R, D))
    dinv_out[...] = dinvb
    y_out[...] = dinvb * xwg


def _kernelB_body(pos_ref, edst_ref, dinv_ref, bgcn_ref, bgat_ref,
                  post_ref, esrct_ref, y_ref, xwa_ref, z1_out, z2_out):
    p = pos_ref[...]
    pb = p.astype(jnp.bfloat16)
    p2i = jnp.sum(p * p, axis=1, keepdims=True)

    acc1 = jnp.zeros((R, D), jnp.float32)
    acc2 = [jnp.zeros((R, HC), jnp.float32) for _ in range(NH)]
    ssum = [jnp.zeros((R, 1), jnp.float32) for _ in range(NH)]
    for c in range(NCH):
        d2 = _d2_chunk(pb, p2i, post_ref, c)
        a = jnp.where(d2 <= 36.0, 1.0, 0.0).astype(jnp.bfloat16)
        acc1 = acc1 + lax.dot_general(
            a, y_ref[c * CB:(c + 1) * CB, :], (((1,), (0,)), ((), ())),
            preferred_element_type=jnp.float32)
        mm = d2 <= 100.0
        for h in range(NH):
            eai = edst_ref[:, h:h + 1]
            ea2i = edst_ref[:, 4 + h:5 + h]
            ebj = esrct_ref[h:h + 1, c * CB:(c + 1) * CB]
            eb2j = esrct_ref[4 + h:5 + h, c * CB:(c + 1) * CB]
            ex = jnp.where(mm, jnp.maximum(eai * ebj, ea2i * eb2j), 0.0)
            ssum[h] = ssum[h] + jnp.sum(ex, axis=1, keepdims=True)
            acc2[h] = acc2[h] + lax.dot_general(
                ex.astype(jnp.bfloat16),
                xwa_ref[c * CB:(c + 1) * CB, h * HC:(h + 1) * HC],
                (((1,), (0,)), ((), ())),
                preferred_element_type=jnp.float32)
    z1_out[...] = _elu(dinv_ref[...] * acc1 + bgcn_ref[0:1, :])
    parts = [acc2[h] / (ssum[h] + 1e-16) for h in range(NH)]
    z2_out[...] = _elu(jnp.concatenate(parts, axis=1) + bgat_ref[0:1, :])


def _kernelC_body(z1_ref, z2_ref, xsum_ref, x_ref, wl_ref, wr_ref, bl_ref,
                  wg_ref, bg_ref, gamma_ref, beta_ref, z3_out, zn_out):
    mean = xsum_ref[...] / 20.0
    z3 = _elu(_dot(mean, wl_ref[...]) + bl_ref[0:1, :]
              + _dot(x_ref[...], wr_ref[...]))
    z3_out[...] = z3
    z1 = z1_ref[...]
    z2 = z2_ref[...]
    logits = (_dot(z1, wg_ref[0:D, :]) + _dot(z2, wg_ref[D:2 * D, :])
              + _dot(z3, wg_ref[2 * D:3 * D, :]) + bg_ref[0:1, :])
    lane = lax.broadcasted_iota(jnp.int32, logits.shape, 1)
    lm = jnp.max(jnp.where(lane < 3, logits, -1e30), axis=1, keepdims=True)
    exl = jnp.where(lane < 3, jnp.exp(logits - lm), 0.0)
    g = exl / jnp.sum(exl, axis=1, keepdims=True)
    zf = g[:, 0:1] * z1 + g[:, 1:2] * z2 + g[:, 2:3] * z3
    mu = jnp.mean(zf, axis=1, keepdims=True)
    var = jnp.mean((zf - mu) ** 2, axis=1, keepdims=True)
    zn_out[...] = ((zf - mu) / jnp.sqrt(var + 1e-5) * gamma_ref[0:1, :]
                   + beta_ref[0:1, :])


def _build_sc_gather():
    mesh = plsc.VectorSubcoreMesh(core_axis_name="c", subcore_axis_name="s")

    @functools.partial(
        pl.kernel, mesh=mesh,
        out_type=jax.ShapeDtypeStruct((NSC, D), jnp.float32),
        scratch_types=[
            pltpu.VMEM((SC_CH * 20,), jnp.int32),
            pltpu.VMEM((SC_CH * 20, D), jnp.float32),
            pltpu.VMEM((SC_CH * SC_G, D), jnp.float32),
            pltpu.SemaphoreType.DMA,
        ],
    )
    def sc_gather(nbr_hbm, x_hbm, out_hbm, idx_v, rows_v, acc_v, sem):
        wid = lax.axis_index("s") * 2 + lax.axis_index("c")

        def group_body(grp, _):
            gbase = wid * SC_PER_W + grp * (SC_CH * SC_G)
            for g in range(SC_G):
                base = gbase + g * SC_CH
                pltpu.sync_copy(nbr_hbm.at[pl.ds(base * 20, SC_CH * 20)],
                                idx_v)
                pltpu.async_copy(x_hbm.at[idx_v], rows_v, sem).wait()

                def node_body(n, __):
                    for cc in range(D // 16):
                        sl = pl.ds(cc * 16, 16)
                        acc = rows_v[n * 20, sl]
                        for k in range(1, 20):
                            acc = acc + rows_v[n * 20 + k, sl]
                        acc_v[g * SC_CH + n, sl] = acc
                    return __

                lax.fori_loop(0, SC_CH, node_body, 0)
            pltpu.sync_copy(acc_v, out_hbm.at[pl.ds(gbase, SC_CH * SC_G)])
            return _

        lax.fori_loop(0, SC_GROUPS, group_body, 0)

    return sc_gather


_SC_GATHER = _build_sc_gather()


def _att_matrix(att, negslope=0.2):
    eye = jnp.eye(NH, dtype=jnp.float32)
    blk = (att[:, :, None] * eye[:, None, :]).reshape(D, NH)
    return jnp.concatenate([blk, negslope * blk], axis=1)


def kernel(x, pos, W_gcn, b_gcn, W_gat, att_src, att_dst, b_gat, W_sage_l,
           b_sage_l, W_sage_r, W_gate, b_gate, gamma, beta):
    f32 = jnp.float32
    posT = jnp.pad(pos.T, ((0, 5), (0, NPAD - N)),
                   constant_values=1024.0).astype(jnp.bfloat16)
    asrc_m = _att_matrix(att_src)
    adst_m = _att_matrix(att_dst)

    full = lambda shape: pl.BlockSpec(shape, lambda i: tuple(0 for _ in shape))
    rblk = lambda d: pl.BlockSpec((R, d), lambda i: (i, 0))

    nbr, y, dinv, xwa, esrc, edst = _PCALL(
        _kernelA_body,
        grid=GRID,
        in_specs=[rblk(D), rblk(3), full((8, NPAD)), full((D, D)),
                  full((D, D)), full((D, 8)), full((D, 8))],
        out_specs=[rblk(32), rblk(D), rblk(D), rblk(D), rblk(8), rblk(8)],
        out_shape=[
            jax.ShapeDtypeStruct((N, 32), jnp.int32),
            jax.ShapeDtypeStruct((N, D), f32),
            jax.ShapeDtypeStruct((N, D), f32),
            jax.ShapeDtypeStruct((N, D), f32),
            jax.ShapeDtypeStruct((N, 8), f32),
            jax.ShapeDtypeStruct((N, 8), f32),
        ],
    )(x, pos, posT, W_gcn, W_gat, asrc_m, adst_m)

    nbr_flat = jnp.pad(nbr[:, :20], ((0, NSC - N), (0, 0))).reshape(-1)
    xsum = _SC_GATHER(nbr_flat, x)[:N]

    ypad = jnp.pad(y, ((0, NPAD - N), (0, 0))).astype(jnp.bfloat16)
    xwapad = jnp.pad(xwa, ((0, NPAD - N), (0, 0))).astype(jnp.bfloat16)
    esrcT = jnp.pad(esrc.T, ((0, 0), (0, NPAD - N)))

    bgcn8 = jnp.broadcast_to(b_gcn[None, :], (8, D))
    bgat8 = jnp.broadcast_to(b_gat[None, :], (8, D))

    z1, z2 = _PCALL(
        _kernelB_body,
        grid=GRID,
        in_specs=[rblk(3), rblk(8), rblk(D), full((8, D)), full((8, D)),
                  full((8, NPAD)), full((8, NPAD)), full((NPAD, D)),
                  full((NPAD, D))],
        out_specs=[rblk(D), rblk(D)],
        out_shape=[jax.ShapeDtypeStruct((N, D), f32),
                   jax.ShapeDtypeStruct((N, D), f32)],
    )(pos, edst, dinv, bgcn8, bgat8, posT, esrcT, ypad, xwapad)

    bl8 = jnp.broadcast_to(b_sage_l[None, :], (8, D))
    wg_pad = jnp.pad(W_gate, ((0, 0), (0, D - 3)))
    bg8 = jnp.broadcast_to(jnp.pad(b_gate, (0, D - 3))[None, :], (8, D))
    gamma8 = jnp.broadcast_to(gamma[None, :], (8, D))
    beta8 = jnp.broadcast_to(beta[None, :], (8, D))

    z3, zn = _PCALL(
        _kernelC_body,
        grid=GRID,
        in_specs=[rblk(D), rblk(D), rblk(D), rblk(D), full((D, D)),
                  full((D, D)), full((8, D)), full((3 * D, D)),
                  full((8, D)), full((8, D)), full((8, D))],
        out_specs=[rblk(D), rblk(D)],
        out_shape=[jax.ShapeDtypeStruct((N, D), f32),
                   jax.ShapeDtypeStruct((N, D), f32)],
    )(z1, z2, xsum, x, W_sage_l, W_sage_r, bl8, wg_pad, bg8, gamma8, beta8)

    return ((z1, z2, z3), zn)


# speedup vs baseline: 2.5078x; 1.8229x over previous
import functools

import jax
import jax.numpy as jnp
from jax import lax
from jax.experimental import pallas as pl
from jax.experimental.pallas import tpu as pltpu
from jax.experimental.pallas import tpu_sc as plsc

_PCALL = pl.pallas_call

N = 10000
D = 128
NH = 4
HC = 32
R = 256
CB = 2048
NPAD = 10240
NCH = NPAD // CB
GRID = (N + R - 1) // R
IMAX = 2**31 - 1

SC_W = 32
SC_CH = 6
SC_G = 4
SC_GROUPS = 14
SC_PER_W = SC_CH * SC_G * SC_GROUPS
NSC = SC_W * SC_PER_W

_HIGH = jax.lax.Precision.HIGHEST


def _dot(a, b):
    return lax.dot_general(a, b, (((1,), (0,)), ((), ())),
                           preferred_element_type=jnp.float32,
                           precision=_HIGH)


def _elu(v):
    return jnp.where(v > 0.0, v, jnp.exp(v) - 1.0)


def _d2_chunk(pb, p2i, post_ref, c):
    pj = post_ref[0:3, c * CB:(c + 1) * CB]
    pjf = pj.astype(jnp.float32)
    p2j = jnp.sum(pjf * pjf, axis=0, keepdims=True)
    dot = lax.dot_general(pb, pj, (((1,), (0,)), ((), ())),
                          preferred_element_type=jnp.float32)
    return p2i + p2j - 2.0 * dot


def _extract_topk(key, k, width):
    buf = jnp.full((key.shape[0], width), IMAX, jnp.int32)
    lane = lax.broadcasted_iota(jnp.int32, (key.shape[0], width), 1)
    for t in range(k):
        m = jnp.min(key, axis=1, keepdims=True)
        buf = jnp.where(lane == t, m, buf)
        key = jnp.where(key == m, IMAX, key)
    return buf


def _kernelA_body(x_ref, pos_ref, post_ref, wg_ref, wa_ref, asrc_ref,
                  adst_ref, nbr_out, y_out, dinv_out, xwa_out, esrc_out,
                  edst_out, key_ref):
    i = pl.program_id(0)
    x = x_ref[...]
    xwg = _dot(x, wg_ref[...])
    xwa = _dot(x, wa_ref[...])
    xwa_out[...] = xwa
    esrc_out[...] = jnp.exp(_dot(xwa, asrc_ref[...]))
    edst_out[...] = jnp.exp(_dot(xwa, adst_ref[...]))

    p = pos_ref[...]
    pb = p.astype(jnp.bfloat16)
    p2i = jnp.sum(p * p, axis=1, keepdims=True)
    rows = i * R + lax.broadcasted_iota(jnp.int32, (R, 1), 0)

    deg = jnp.zeros((R, 1), jnp.float32)
    m = [jnp.full((R, 128), IMAX, jnp.int32) for _ in range(5)]
    for c in range(NCH):
        d2 = _d2_chunk(pb, p2i, post_ref, c)
        deg = deg + jnp.sum(jnp.where(d2 <= 36.0, 1.0, 0.0),
                            axis=1, keepdims=True)
        jj = c * CB + lax.broadcasted_iota(jnp.int32, (R, CB), 1)
        valid = (d2 < 32768.0) & (jj != rows)
        key = jnp.where(valid, d2.astype(jnp.int32) * 16384 + jj, IMAX)
        key_ref[:, c * CB:(c + 1) * CB] = key
        for s in range(CB // 128):
            v = key[:, s * 128:(s + 1) * 128]
            b = [v < mk for mk in m]
            m = [jnp.where(b[0], v, m[0]),
                 jnp.where(b[0], m[0], jnp.where(b[1], v, m[1])),
                 jnp.where(b[1], m[1], jnp.where(b[2], v, m[2])),
                 jnp.where(b[2], m[2], jnp.where(b[3], v, m[3])),
                 jnp.where(b[3], m[3], jnp.where(b[4], v, m[4]))]

    top = _extract_topk(jnp.concatenate(m, axis=1), 20, 32)
    t20 = top[:, 19:20]
    cnt = jnp.zeros((R, 1), jnp.int32)
    for c in range(NCH):
        k = key_ref[:, c * CB:(c + 1) * CB]
        cnt = cnt + jnp.sum(jnp.where(k <= t20, 1, 0), axis=1,
                            keepdims=True)
    bad = jnp.sum(jnp.where(cnt != 20, 1, 0)) > 0

    def _full():
        bs = [_extract_topk(key_ref[:, c * CB:(c + 1) * CB], 20, 32)
              for c in range(NCH)]
        return _extract_topk(jnp.concatenate(bs, axis=1), 20, 32)

    top = lax.cond(bad, _full, lambda: top)
    lane = lax.broadcasted_iota(jnp.int32, (R, 32), 1)
    nbr_out[...] = jnp.where(lane < 20, jnp.bitwise_and(top, 16383), 0)

    dinv = lax.rsqrt(deg)
    dinvb = jnp.broadcast_to(dinv, (R, D))
    dinv_out[...] = dinvb
    y_out[...] = dinvb * xwg


def _kernelB_body(pos_ref, edst_ref, dinv_ref, bgcn_ref, bgat_ref,
                  post_ref, esrct_ref, y_ref, xwa_ref, z1_out, z2_out):
    p = pos_ref[...]
    pb = p.astype(jnp.bfloat16)
    p2i = jnp.sum(p * p, axis=1, keepdims=True)

    acc1 = jnp.zeros((R, D), jnp.float32)
    acc2 = [jnp.zeros((R, HC), jnp.float32) for _ in range(NH)]
    ssum = [jnp.zeros((R, 1), jnp.float32) for _ in range(NH)]
    for c in range(NCH):
        d2 = _d2_chunk(pb, p2i, post_ref, c)
        a = jnp.where(d2 <= 36.0, 1.0, 0.0).astype(jnp.bfloat16)
        acc1 = acc1 + lax.dot_general(
            a, y_ref[c * CB:(c + 1) * CB, :], (((1,), (0,)), ((), ())),
            preferred_element_type=jnp.float32)
        mm = d2 <= 100.0
        for h in range(NH):
            eai = edst_ref[:, h:h + 1]
            ea2i = edst_ref[:, 4 + h:5 + h]
            ebj = esrct_ref[h:h + 1, c * CB:(c + 1) * CB]
            eb2j = esrct_ref[4 + h:5 + h, c * CB:(c + 1) * CB]
            ex = jnp.where(mm, jnp.maximum(eai * ebj, ea2i * eb2j), 0.0)
            ssum[h] = ssum[h] + jnp.sum(ex, axis=1, keepdims=True)
            acc2[h] = acc2[h] + lax.dot_general(
                ex.astype(jnp.bfloat16),
                xwa_ref[c * CB:(c + 1) * CB, h * HC:(h + 1) * HC],
                (((1,), (0,)), ((), ())),
                preferred_element_type=jnp.float32)
    z1_out[...] = _elu(dinv_ref[...] * acc1 + bgcn_ref[0:1, :])
    parts = [acc2[h] / (ssum[h] + 1e-16) for h in range(NH)]
    z2_out[...] = _elu(jnp.concatenate(parts, axis=1) + bgat_ref[0:1, :])


def _kernelC_body(z1_ref, z2_ref, xsum_ref, x_ref, wl_ref, wr_ref, bl_ref,
                  wg_ref, bg_ref, gamma_ref, beta_ref, z3_out, zn_out):
    mean = xsum_ref[...] / 20.0
    z3 = _elu(_dot(mean, wl_ref[...]) + bl_ref[0:1, :]
              + _dot(x_ref[...], wr_ref[...]))
    z3_out[...] = z3
    z1 = z1_ref[...]
    z2 = z2_ref[...]
    logits = (_dot(z1, wg_ref[0:D, :]) + _dot(z2, wg_ref[D:2 * D, :])
              + _dot(z3, wg_ref[2 * D:3 * D, :]) + bg_ref[0:1, :])
    lane = lax.broadcasted_iota(jnp.int32, logits.shape, 1)
    lm = jnp.max(jnp.where(lane < 3, logits, -1e30), axis=1, keepdims=True)
    exl = jnp.where(lane < 3, jnp.exp(logits - lm), 0.0)
    g = exl / jnp.sum(exl, axis=1, keepdims=True)
    zf = g[:, 0:1] * z1 + g[:, 1:2] * z2 + g[:, 2:3] * z3
    mu = jnp.mean(zf, axis=1, keepdims=True)
    var = jnp.mean((zf - mu) ** 2, axis=1, keepdims=True)
    zn_out[...] = ((zf - mu) / jnp.sqrt(var + 1e-5) * gamma_ref[0:1, :]
                   + beta_ref[0:1, :])


def _build_sc_gather():
    mesh = plsc.VectorSubcoreMesh(core_axis_name="c", subcore_axis_name="s")

    @functools.partial(
        pl.kernel, mesh=mesh,
        out_type=jax.ShapeDtypeStruct((NSC, D), jnp.float32),
        scratch_types=[
            pltpu.VMEM((SC_CH * 20,), jnp.int32),
            pltpu.VMEM((SC_CH * 20, D), jnp.float32),
            pltpu.VMEM((SC_CH * SC_G, D), jnp.float32),
            pltpu.SemaphoreType.DMA,
        ],
    )
    def sc_gather(nbr_hbm, x_hbm, out_hbm, idx_v, rows_v, acc_v, sem):
        wid = lax.axis_index("s") * 2 + lax.axis_index("c")

        def group_body(grp, _):
            gbase = wid * SC_PER_W + grp * (SC_CH * SC_G)
            for g in range(SC_G):
                base = gbase + g * SC_CH
                pltpu.sync_copy(nbr_hbm.at[pl.ds(base * 20, SC_CH * 20)],
                                idx_v)
                pltpu.async_copy(x_hbm.at[idx_v], rows_v, sem).wait()

                def node_body(n, __):
                    for cc in range(D // 16):
                        sl = pl.ds(cc * 16, 16)
                        acc = rows_v[n * 20, sl]
                        for k in range(1, 20):
                            acc = acc + rows_v[n * 20 + k, sl]
                        acc_v[g * SC_CH + n, sl] = acc
                    return __

                lax.fori_loop(0, SC_CH, node_body, 0)
            pltpu.sync_copy(acc_v, out_hbm.at[pl.ds(gbase, SC_CH * SC_G)])
            return _

        lax.fori_loop(0, SC_GROUPS, group_body, 0)

    return sc_gather


_SC_GATHER = _build_sc_gather()


def _att_matrix(att, negslope=0.2):
    eye = jnp.eye(NH, dtype=jnp.float32)
    blk = (att[:, :, None] * eye[:, None, :]).reshape(D, NH)
    return jnp.concatenate([blk, negslope * blk], axis=1)


def kernel(x, pos, W_gcn, b_gcn, W_gat, att_src, att_dst, b_gat, W_sage_l,
           b_sage_l, W_sage_r, W_gate, b_gate, gamma, beta):
    f32 = jnp.float32
    posT = jnp.pad(pos.T, ((0, 5), (0, NPAD - N)),
                   constant_values=1024.0).astype(jnp.bfloat16)
    asrc_m = _att_matrix(att_src)
    adst_m = _att_matrix(att_dst)

    full = lambda shape: pl.BlockSpec(shape, lambda i: tuple(0 for _ in shape))
    rblk = lambda d: pl.BlockSpec((R, d), lambda i: (i, 0))

    nbr, y, dinv, xwa, esrc, edst = _PCALL(
        _kernelA_body,
        grid=GRID,
        in_specs=[rblk(D), rblk(3), full((8, NPAD)), full((D, D)),
                  full((D, D)), full((D, 8)), full((D, 8))],
        out_specs=[rblk(32), rblk(D), rblk(D), rblk(D), rblk(8), rblk(8)],
        out_shape=[
            jax.ShapeDtypeStruct((N, 32), jnp.int32),
            jax.ShapeDtypeStruct((N, D), f32),
            jax.ShapeDtypeStruct((N, D), f32),
            jax.ShapeDtypeStruct((N, D), f32),
            jax.ShapeDtypeStruct((N, 8), f32),
            jax.ShapeDtypeStruct((N, 8), f32),
        ],
        scratch_shapes=[pltpu.VMEM((R, NPAD), jnp.int32)],
    )(x, pos, posT, W_gcn, W_gat, asrc_m, adst_m)

    nbr_flat = jnp.pad(nbr[:, :20], ((0, NSC - N), (0, 0))).reshape(-1)
    xsum = _SC_GATHER(nbr_flat, x)[:N]

    ypad = jnp.pad(y, ((0, NPAD - N), (0, 0))).astype(jnp.bfloat16)
    xwapad = jnp.pad(xwa, ((0, NPAD - N), (0, 0))).astype(jnp.bfloat16)
    esrcT = jnp.pad(esrc.T, ((0, 0), (0, NPAD - N)))

    bgcn8 = jnp.broadcast_to(b_gcn[None, :], (8, D))
    bgat8 = jnp.broadcast_to(b_gat[None, :], (8, D))

    z1, z2 = _PCALL(
        _kernelB_body,
        grid=GRID,
        in_specs=[rblk(3), rblk(8), rblk(D), full((8, D)), full((8, D)),
                  full((8, NPAD)), full((8, NPAD)), full((NPAD, D)),
                  full((NPAD, D))],
        out_specs=[rblk(D), rblk(D)],
        out_shape=[jax.ShapeDtypeStruct((N, D), f32),
                   jax.ShapeDtypeStruct((N, D), f32)],
    )(pos, edst, dinv, bgcn8, bgat8, posT, esrcT, ypad, xwapad)

    bl8 = jnp.broadcast_to(b_sage_l[None, :], (8, D))
    wg_pad = jnp.pad(W_gate, ((0, 0), (0, D - 3)))
    bg8 = jnp.broadcast_to(jnp.pad(b_gate, (0, D - 3))[None, :], (8, D))
    gamma8 = jnp.broadcast_to(gamma[None, :], (8, D))
    beta8 = jnp.broadcast_to(beta[None, :], (8, D))

    z3, zn = _PCALL(
        _kernelC_body,
        grid=GRID,
        in_specs=[rblk(D), rblk(D), rblk(D), rblk(D), full((D, D)),
                  full((D, D)), full((8, D)), full((3 * D, D)),
                  full((8, D)), full((8, D)), full((8, D))],
        out_specs=[rblk(D), rblk(D)],
        out_shape=[jax.ShapeDtypeStruct((N, D), f32),
                   jax.ShapeDtypeStruct((N, D), f32)],
    )(z1, z2, xsum, x, W_sage_l, W_sage_r, bl8, wg_pad, bg8, gamma8, beta8)

    return ((z1, z2, z3), zn)
